# Initial kernel scaffold; baseline (speedup 1.0000x reference)
import jax
import jax.numpy as jnp
from jax.experimental import pallas as pl


def kernel(input_ids, token_embedding):
    raise NotImplementedError("write your pallas kernel here")


# speedup vs baseline: 1.4443x; 1.4443x over previous
import functools

import jax
import jax.numpy as jnp
from jax import lax
from jax.experimental import pallas as pl
from jax.experimental.pallas import tpu as pltpu
from jax.experimental.pallas import tpu_sc as plsc


@functools.lru_cache(maxsize=None)
def _build_gather(vocab, dim, n_rows):
    info = plsc.get_sparse_core_info()
    nc, ns = info.num_cores, info.num_subcores
    nw = nc * ns
    rows_per_w = n_rows // nw
    chunk = 16
    n_chunks = rows_per_w // chunk

    mesh = plsc.VectorSubcoreMesh(core_axis_name="c", subcore_axis_name="s")

    @functools.partial(
        pl.kernel,
        mesh=mesh,
        out_type=jax.ShapeDtypeStruct((n_rows, dim), jnp.float32),
        scratch_types=[
            pltpu.VMEM((n_chunks, chunk), jnp.int32),
            pltpu.VMEM((chunk, dim), jnp.float32),
            pltpu.SemaphoreType.DMA,
        ],
    )
    def gather_kernel(idx_hbm, table_hbm, out_hbm, idx_v, buf, sem):
        wid = lax.axis_index("s") * nc + lax.axis_index("c")
        base = wid * rows_per_w
        pltpu.sync_copy(idx_hbm.at[wid], idx_v)

        def body(i, carry):
            pltpu.async_copy(table_hbm.at[idx_v.at[i]], buf, sem).wait()
            pltpu.sync_copy(buf, out_hbm.at[pl.ds(base + i * chunk, chunk)])
            return carry

        lax.fori_loop(0, n_chunks, body, 0)

    return gather_kernel, nw, n_chunks, chunk


def kernel(input_ids, token_embedding):
    b, s = input_ids.shape
    vocab, dim = token_embedding.shape
    n_rows = b * s
    fn, nw, n_chunks, chunk = _build_gather(vocab, dim, n_rows)
    idx = input_ids.reshape(nw, n_chunks, chunk)
    out = fn(idx, token_embedding)
    return out.reshape(b, s, dim)


# speedup vs baseline: 1.5850x; 1.0974x over previous
import functools

import jax
import jax.numpy as jnp
from jax import lax
from jax.experimental import pallas as pl
from jax.experimental.pallas import tpu as pltpu
from jax.experimental.pallas import tpu_sc as plsc


@functools.lru_cache(maxsize=None)
def _build_gather(vocab, dim, n_rows):
    info = plsc.get_sparse_core_info()
    nc, ns = info.num_cores, info.num_subcores
    nw = nc * ns
    rows_per_w = n_rows // nw
    chunk = 16
    n_chunks = rows_per_w // chunk

    mesh = plsc.VectorSubcoreMesh(core_axis_name="c", subcore_axis_name="s")

    @functools.partial(
        pl.kernel,
        mesh=mesh,
        out_type=jax.ShapeDtypeStruct((n_rows, dim), jnp.float32),
        scratch_types=[
            pltpu.VMEM((n_chunks, chunk), jnp.int32),
            pltpu.VMEM((chunk, dim), jnp.float32),
            pltpu.VMEM((chunk, dim), jnp.float32),
            pltpu.SemaphoreType.DMA,
            pltpu.SemaphoreType.DMA,
            pltpu.SemaphoreType.DMA,
            pltpu.SemaphoreType.DMA,
        ],
    )
    def gather_kernel(idx_hbm, table_hbm, out_hbm, idx_v, buf0, buf1,
                      gsem0, gsem1, ssem0, ssem1):
        wid = lax.axis_index("s") * nc + lax.axis_index("c")
        base = wid * rows_per_w
        pltpu.sync_copy(idx_hbm.at[wid], idx_v)

        bufs = (buf0, buf1)
        gsems = (gsem0, gsem1)
        ssems = (ssem0, ssem1)

        g_handles = [None] * n_chunks
        s_handles = [None] * n_chunks
        g_handles[0] = pltpu.async_copy(table_hbm.at[idx_v.at[0]], buf0, gsem0)
        for i in range(n_chunks):
            b = i % 2
            g_handles[i].wait()
            s_handles[i] = pltpu.async_copy(
                bufs[b], out_hbm.at[pl.ds(base + i * chunk, chunk)], ssems[b])
            if i + 1 < n_chunks:
                if i >= 1:
                    s_handles[i - 1].wait()
                g_handles[i + 1] = pltpu.async_copy(
                    table_hbm.at[idx_v.at[i + 1]], bufs[1 - b], gsems[1 - b])
        s_handles[n_chunks - 2].wait()
        s_handles[n_chunks - 1].wait()

    return gather_kernel, nw, n_chunks, chunk


def kernel(input_ids, token_embedding):
    b, s = input_ids.shape
    vocab, dim = token_embedding.shape
    n_rows = b * s
    fn, nw, n_chunks, chunk = _build_gather(vocab, dim, n_rows)
    idx = input_ids.reshape(nw, n_chunks, chunk)
    out = fn(idx, token_embedding)
    return out.reshape(b, s, dim)


# speedup vs baseline: 1.6704x; 1.0539x over previous
import functools

import jax
import jax.numpy as jnp
from jax import lax
from jax.experimental import pallas as pl
from jax.experimental.pallas import tpu as pltpu
from jax.experimental.pallas import tpu_sc as plsc


@functools.lru_cache(maxsize=None)
def _build_gather(vocab, dim, n_rows):
    info = plsc.get_sparse_core_info()
    nc, ns = info.num_cores, info.num_subcores
    nw = nc * ns
    rows_per_w = n_rows // nw
    chunk = 16
    n_chunks = rows_per_w // chunk
    nb = 3

    mesh = plsc.VectorSubcoreMesh(core_axis_name="c", subcore_axis_name="s")

    @functools.partial(
        pl.kernel,
        mesh=mesh,
        out_type=jax.ShapeDtypeStruct((n_rows, dim), jnp.float32),
        scratch_types=[
            pltpu.VMEM((n_chunks, chunk), jnp.int32),
        ]
        + [pltpu.VMEM((chunk, dim), jnp.float32) for _ in range(nb)]
        + [pltpu.SemaphoreType.DMA for _ in range(2 * nb)],
    )
    def gather_kernel(idx_hbm, table_hbm, out_hbm, idx_v, *rest):
        bufs = rest[:nb]
        gsems = rest[nb:2 * nb]
        ssems = rest[2 * nb:]
        wid = lax.axis_index("s") * nc + lax.axis_index("c")
        base = wid * rows_per_w
        pltpu.sync_copy(idx_hbm.at[wid], idx_v)

        g_handles = [None] * n_chunks
        s_handles = [None] * n_chunks
        for j in range(min(nb, n_chunks)):
            g_handles[j] = pltpu.async_copy(
                table_hbm.at[idx_v.at[j]], bufs[j], gsems[j])
        for i in range(n_chunks):
            if i >= 1 and i + nb - 1 < n_chunks:
                s_handles[i - 1].wait()
                j = i + nb - 1
                g_handles[j] = pltpu.async_copy(
                    table_hbm.at[idx_v.at[j]], bufs[j % nb], gsems[j % nb])
            g_handles[i].wait()
            s_handles[i] = pltpu.async_copy(
                bufs[i % nb], out_hbm.at[pl.ds(base + i * chunk, chunk)],
                ssems[i % nb])
        for i in range(max(0, n_chunks - nb), n_chunks):
            s_handles[i].wait()

    return gather_kernel, nw, n_chunks, chunk


def kernel(input_ids, token_embedding):
    b, s = input_ids.shape
    vocab, dim = token_embedding.shape
    n_rows = b * s
    fn, nw, n_chunks, chunk = _build_gather(vocab, dim, n_rows)
    idx = input_ids.reshape(nw, n_chunks, chunk)
    out = fn(idx, token_embedding)
    return out.reshape(b, s, dim)
